# Initial kernel scaffold; baseline (speedup 1.0000x reference)
import jax
import jax.numpy as jnp
from jax.experimental import pallas as pl


def kernel(logits, targets):
    raise NotImplementedError("write your pallas kernel here")


# speedup vs baseline: 26.8483x; 26.8483x over previous
import jax
import jax.numpy as jnp
from jax import lax
from jax.experimental import pallas as pl
from jax.experimental.pallas import tpu as pltpu

_TAU = 1.5
_BIG = 1.0e9


def _loss_body(logits_t_ref, targets_ref, out_ref, m_ref, gt_ref):
    B, C, W, H = logits_t_ref.shape
    NC = C - 1
    BC = B * NC

    targets = targets_ref[...]
    cls = (lax.broadcasted_iota(jnp.int32, (BC, 1, 1), 0) % NC) + 1
    tgt_rep = jnp.broadcast_to(targets[:, None, :, :], (B, NC, H, W)).reshape(BC, H, W)
    m_ref[...] = jnp.where(tgt_rep == cls, 0.0, _BIG).astype(jnp.float32)

    yy = lax.broadcasted_iota(jnp.int32, (H,), 0).astype(jnp.float32)

    def s1(yt, Gt):
        mrow = m_ref[:, yt, :]
        dy = (yy - yt.astype(jnp.float32)) ** 2
        return jnp.minimum(Gt, mrow[:, :, None] + dy[None, None, :])

    gt_ref[...] = lax.fori_loop(0, H, s1, jnp.full((BC, W, H), _BIG, jnp.float32))

    xx = lax.broadcasted_iota(jnp.int32, (W,), 0).astype(jnp.float32)

    def s2(x, D2):
        grow = gt_ref[:, x, :]
        dx = (xx - x.astype(jnp.float32)) ** 2
        return jnp.minimum(D2, grow[:, None, :] + dx[None, :, None])

    D2 = lax.fori_loop(0, W, s2, jnp.full((BC, W, H), _BIG, jnp.float32))

    R = jnp.exp(-jnp.sqrt(D2) / _TAU).reshape(B, NC, W, H)

    lt = logits_t_ref[...]
    m = jnp.max(lt, axis=1, keepdims=True)
    e = jnp.exp(lt - m)
    denom = jnp.sum(e, axis=1)
    num = jnp.sum(e[:, 1:, :, :] * R, axis=1)
    out_ref[...] = (-jnp.sum(num / denom) / (B * H * W)).reshape(1, 1)


def kernel(logits, targets):
    B, C, H, W = logits.shape
    logits_t = jnp.swapaxes(logits, 2, 3)
    NC = C - 1
    out = pl.pallas_call(
        _loss_body,
        out_shape=jax.ShapeDtypeStruct((1, 1), jnp.float32),
        scratch_shapes=[
            pltpu.VMEM((B * NC, H, W), jnp.float32),
            pltpu.VMEM((B * NC, W, H), jnp.float32),
        ],
    )(logits_t, targets)
    return out[0, 0]


# speedup vs baseline: 41.6550x; 1.5515x over previous
import functools

import jax
import jax.numpy as jnp
from jax import lax
from jax.experimental import pallas as pl
from jax.experimental.pallas import tpu as pltpu
from jax.experimental.pallas import tpu_sc as plsc

_TAU = 1.5
_B, _C, _H, _W = 8, 10, 96, 96
_NCLS = _C - 1
_NW = 32
_RB = _H // 4
_LANES = 16
_KX = _W // _LANES
_D2MAX = 2 * (_H - 1) * (_H - 1)
_TABN = ((_D2MAX + 2 + 7) // 8) * 8
_FAR = 1024
_ACC0 = 1 << 22


def _splat_i32(x):
    return jnp.full((_LANES,), x, dtype=jnp.int32)


def _sc_body(logits_hbm, targets_hbm, dx2_hbm, tab_hbm, out_hbm,
             tgt_v, log_v, dx2_v, tab_v, d1sq_v, maxl_v, denom_v, num_v,
             out_v):
    wid = lax.axis_index("s") * 2 + lax.axis_index("c")
    b = wid // 4
    row0 = (wid % 4) * _RB

    pltpu.sync_copy(targets_hbm.at[b], tgt_v)
    pltpu.sync_copy(logits_hbm.at[b, :, pl.ds(row0, _RB), :], log_v)
    pltpu.sync_copy(dx2_hbm, dx2_v)
    pltpu.sync_copy(tab_hbm, tab_v)

    for c in range(1, _C):
        def fwd(y, dist):
            new = []
            for k in range(_KX):
                lbl = tgt_v[y, pl.ds(k * _LANES, _LANES)]
                new.append(jnp.where(lbl == c, 0, dist[k] + 1))
            rel = y - row0

            @pl.when(jnp.logical_and(rel >= 0, rel < _RB))
            def _():
                for k in range(_KX):
                    off = ((c - 1) * _RB + rel) * _W + k * _LANES
                    d1sq_v[pl.ds(off, _LANES)] = new[k]

            return tuple(new)

        lax.fori_loop(0, _H, fwd, tuple(_splat_i32(_FAR) for _ in range(_KX)))

        def bwd(i, dist):
            y = (_H - 1) - i
            new = []
            for k in range(_KX):
                lbl = tgt_v[y, pl.ds(k * _LANES, _LANES)]
                new.append(jnp.where(lbl == c, 0, dist[k] + 1))
            rel = y - row0

            @pl.when(jnp.logical_and(rel >= 0, rel < _RB))
            def _():
                for k in range(_KX):
                    sl = pl.ds(((c - 1) * _RB + rel) * _W + k * _LANES, _LANES)
                    m = jnp.minimum(d1sq_v[sl], new[k])
                    d1sq_v[sl] = m * m

            return tuple(new)

        lax.fori_loop(0, _H, bwd, tuple(_splat_i32(_FAR) for _ in range(_KX)))

    def smax(yq, carry):
        for k in range(_KX):
            sl = pl.ds(k * _LANES, _LANES)
            ls = [log_v[c, yq, sl] for c in range(_C)]
            m = ls[0]
            for l in ls[1:]:
                m = jnp.maximum(m, l)
            s = jnp.zeros((_LANES,), jnp.float32)
            for l in ls:
                s = s + jnp.exp(l - m)
            maxl_v[yq, sl] = m
            denom_v[yq, sl] = s
            num_v[yq, sl] = jnp.zeros((_LANES,), jnp.float32)
        return carry

    lax.fori_loop(0, _RB, smax, 0)

    for c in range(1, _C):
        def row(yq, carry):
            base = ((c - 1) * _RB + yq) * _W

            def xstep(x, accs):
                bc = plsc.load_gather(d1sq_v, [_splat_i32(base + x)])
                return tuple(
                    jnp.minimum(accs[k],
                                bc + dx2_v[x, pl.ds(k * _LANES, _LANES)])
                    for k in range(_KX))

            accs = lax.fori_loop(
                0, _W, xstep, tuple(_splat_i32(_ACC0) for _ in range(_KX)),
                unroll=4)
            for k in range(_KX):
                sl = pl.ds(k * _LANES, _LANES)
                idx = jnp.minimum(accs[k], _D2MAX + 1)
                rew = plsc.load_gather(tab_v, [idx])
                e = jnp.exp(log_v[c, yq, sl] - maxl_v[yq, sl])
                num_v[yq, sl] = num_v[yq, sl] + e * rew
            return carry

        lax.fori_loop(0, _RB, row, 0)

    def fin(yq, accs):
        return tuple(
            accs[k] + num_v[yq, pl.ds(k * _LANES, _LANES)]
            / denom_v[yq, pl.ds(k * _LANES, _LANES)]
            for k in range(_KX))

    accs = lax.fori_loop(0, _RB, fin,
                         tuple(jnp.zeros((_LANES,), jnp.float32)
                               for _ in range(_KX)))
    tot = accs[0]
    for k in range(1, _KX):
        tot = tot + accs[k]
    out_v[...] = tot
    pltpu.sync_copy(out_v, out_hbm.at[wid])


@jax.jit
def kernel(logits, targets):
    i = jnp.arange(_TABN)
    tab = jnp.where(i <= _D2MAX,
                    jnp.exp(-jnp.sqrt(i.astype(jnp.float32)) / _TAU),
                    0.0).astype(jnp.float32)
    x = jnp.arange(_W, dtype=jnp.int32)
    dx2 = (x[None, :] - x[:, None]) ** 2

    mesh = plsc.VectorSubcoreMesh(core_axis_name="c", subcore_axis_name="s")
    run = functools.partial(
        pl.kernel, mesh=mesh,
        compiler_params=pltpu.CompilerParams(needs_layout_passes=False),
        out_type=jax.ShapeDtypeStruct((_NW, _LANES), jnp.float32),
        scratch_types=[
            pltpu.VMEM((_H, _W), jnp.int32),
            pltpu.VMEM((_C, _RB, _W), jnp.float32),
            pltpu.VMEM((_W, _W), jnp.int32),
            pltpu.VMEM((_TABN,), jnp.float32),
            pltpu.VMEM((_NCLS * _RB * _W,), jnp.int32),
            pltpu.VMEM((_RB, _W), jnp.float32),
            pltpu.VMEM((_RB, _W), jnp.float32),
            pltpu.VMEM((_RB, _W), jnp.float32),
            pltpu.VMEM((_LANES,), jnp.float32),
        ],
    )(_sc_body)
    partials = run(logits, targets, dx2, tab)
    return -jnp.sum(partials) / (_B * _H * _W)


# speedup vs baseline: 42.3885x; 1.0176x over previous
import functools

import jax
import jax.numpy as jnp
import numpy as np
from jax import lax
from jax.experimental import pallas as pl
from jax.experimental.pallas import tpu as pltpu
from jax.experimental.pallas import tpu_sc as plsc

_TAU = 1.5
_B, _C, _H, _W = 8, 10, 96, 96
_NCLS = _C - 1
_NW = 32
_RB = _H // 4
_LANES = 16
_KX = _W // _LANES
_D2MAX = 2 * (_H - 1) * (_H - 1)
_TABN = ((_D2MAX + 2 + 7) // 8) * 8
_FAR = 1024
_ACC0 = 1 << 22


def _splat_i32(x):
    return jnp.full((_LANES,), x, dtype=jnp.int32)


def _sc_body(logits_hbm, targets_hbm, dx2_hbm, tab_hbm, out_hbm,
             tgt_v, log_v, dx2_v, tab_v, d1sq_v, maxl_v, denom_v, num_v,
             out_v):
    wid = lax.axis_index("s") * 2 + lax.axis_index("c")
    b = wid // 4
    row0 = (wid % 4) * _RB

    pltpu.sync_copy(targets_hbm.at[b], tgt_v)
    pltpu.sync_copy(logits_hbm.at[b, :, pl.ds(row0, _RB), :], log_v)
    pltpu.sync_copy(dx2_hbm, dx2_v)
    pltpu.sync_copy(tab_hbm, tab_v)

    for c in range(1, _C):
        def fwd(y, dist):
            new = []
            for k in range(_KX):
                lbl = tgt_v[y, pl.ds(k * _LANES, _LANES)]
                new.append(jnp.where(lbl == c, 0, dist[k] + 1))
            rel = y - row0

            @pl.when(jnp.logical_and(rel >= 0, rel < _RB))
            def _():
                for k in range(_KX):
                    off = ((c - 1) * _RB + rel) * _W + k * _LANES
                    d1sq_v[pl.ds(off, _LANES)] = new[k]

            return tuple(new)

        lax.fori_loop(0, _H, fwd, tuple(_splat_i32(_FAR) for _ in range(_KX)))

        def bwd(i, dist):
            y = (_H - 1) - i
            new = []
            for k in range(_KX):
                lbl = tgt_v[y, pl.ds(k * _LANES, _LANES)]
                new.append(jnp.where(lbl == c, 0, dist[k] + 1))
            rel = y - row0

            @pl.when(jnp.logical_and(rel >= 0, rel < _RB))
            def _():
                for k in range(_KX):
                    sl = pl.ds(((c - 1) * _RB + rel) * _W + k * _LANES, _LANES)
                    m = jnp.minimum(d1sq_v[sl], new[k])
                    d1sq_v[sl] = m * m

            return tuple(new)

        lax.fori_loop(0, _H, bwd, tuple(_splat_i32(_FAR) for _ in range(_KX)))

    def smax(yq, carry):
        for k in range(_KX):
            sl = pl.ds(k * _LANES, _LANES)
            ls = [log_v[c, yq, sl] for c in range(_C)]
            m = ls[0]
            for l in ls[1:]:
                m = jnp.maximum(m, l)
            s = jnp.zeros((_LANES,), jnp.float32)
            for l in ls:
                s = s + jnp.exp(l - m)
            maxl_v[yq, sl] = m
            denom_v[yq, sl] = s
            num_v[yq, sl] = jnp.zeros((_LANES,), jnp.float32)
        return carry

    lax.fori_loop(0, _RB, smax, 0)

    _RG = 3
    for c in range(1, _C):
        def rowgrp(rg, carry):
            yq0 = rg * _RG
            base = ((c - 1) * _RB + yq0) * _W

            def xstep(x, accs):
                bcs = [plsc.load_gather(d1sq_v, [_splat_i32(base + r * _W + x)])
                       for r in range(_RG)]
                out = []
                for r in range(_RG):
                    for k in range(_KX):
                        out.append(jnp.minimum(
                            accs[r * _KX + k],
                            bcs[r] + dx2_v[x, pl.ds(k * _LANES, _LANES)]))
                return tuple(out)

            accs = lax.fori_loop(
                0, _W, xstep,
                tuple(_splat_i32(_ACC0) for _ in range(_RG * _KX)),
                unroll=2)
            for r in range(_RG):
                yq = yq0 + r
                for k in range(_KX):
                    sl = pl.ds(k * _LANES, _LANES)
                    idx = jnp.minimum(accs[r * _KX + k], _D2MAX + 1)
                    rew = plsc.load_gather(tab_v, [idx])
                    e = jnp.exp(log_v[c, yq, sl] - maxl_v[yq, sl])
                    num_v[yq, sl] = num_v[yq, sl] + e * rew
            return carry

        lax.fori_loop(0, _RB // _RG, rowgrp, 0)

    def fin(yq, accs):
        return tuple(
            accs[k] + num_v[yq, pl.ds(k * _LANES, _LANES)]
            / denom_v[yq, pl.ds(k * _LANES, _LANES)]
            for k in range(_KX))

    accs = lax.fori_loop(0, _RB, fin,
                         tuple(jnp.zeros((_LANES,), jnp.float32)
                               for _ in range(_KX)))
    tot = accs[0]
    for k in range(1, _KX):
        tot = tot + accs[k]
    out_v[...] = tot
    pltpu.sync_copy(out_v, out_hbm.at[wid])


_I = np.arange(_TABN)
_TAB_NP = np.where(_I <= _D2MAX, np.exp(-np.sqrt(_I.astype(np.float32)) / _TAU),
                   0.0).astype(np.float32)
_X = np.arange(_W, dtype=np.int32)
_DX2_NP = ((_X[None, :] - _X[:, None]) ** 2).astype(np.int32)


@jax.jit
def kernel(logits, targets):
    tab = jnp.asarray(_TAB_NP)
    dx2 = jnp.asarray(_DX2_NP)

    mesh = plsc.VectorSubcoreMesh(core_axis_name="c", subcore_axis_name="s")
    run = functools.partial(
        pl.kernel, mesh=mesh,
        compiler_params=pltpu.CompilerParams(needs_layout_passes=False),
        out_type=jax.ShapeDtypeStruct((_NW, _LANES), jnp.float32),
        scratch_types=[
            pltpu.VMEM((_H, _W), jnp.int32),
            pltpu.VMEM((_C, _RB, _W), jnp.float32),
            pltpu.VMEM((_W, _W), jnp.int32),
            pltpu.VMEM((_TABN,), jnp.float32),
            pltpu.VMEM((_NCLS * _RB * _W,), jnp.int32),
            pltpu.VMEM((_RB, _W), jnp.float32),
            pltpu.VMEM((_RB, _W), jnp.float32),
            pltpu.VMEM((_RB, _W), jnp.float32),
            pltpu.VMEM((_LANES,), jnp.float32),
        ],
    )(_sc_body)
    partials = run(logits, targets, dx2, tab)
    return -jnp.sum(partials) / (_B * _H * _W)
